# Initial kernel scaffold; baseline (speedup 1.0000x reference)
import jax
import jax.numpy as jnp
from jax.experimental import pallas as pl


def kernel(user_ids, pos_item_ids, user_emb, item_emb, user_bias, item_bias, global_bias, W1, b1, W2, b2, W3, b3):
    raise NotImplementedError("write your pallas kernel here")


# speedup vs baseline: 1.5221x; 1.5221x over previous
import functools

import jax
import jax.numpy as jnp
from jax import lax
from jax.experimental import pallas as pl
from jax.experimental.pallas import tpu as pltpu
from jax.experimental.pallas import tpu_sc as plsc

B = 4096
D = 128
NC = 2
NS = 16
L = 16
NW = NC * NS
BPW = B // NW

_mesh = plsc.VectorSubcoreMesh(core_axis_name="c", subcore_axis_name="s")

_SC_OUT_TYPE = [
    jax.ShapeDtypeStruct((B, D), jnp.float32),
    jax.ShapeDtypeStruct((B,), jnp.float32),
]
_SC_SCRATCH = [
    pltpu.VMEM((BPW,), jnp.int32),
    pltpu.VMEM((BPW,), jnp.int32),
    pltpu.VMEM((BPW, D), jnp.float32),
    pltpu.VMEM((BPW, D), jnp.float32),
    pltpu.VMEM((BPW,), jnp.float32),
    pltpu.VMEM((BPW,), jnp.float32),
    pltpu.SemaphoreType.DMA,
]


def _sc_gather_body(uid_hbm, iid_hbm, uemb_hbm, iemb_hbm, ubias_hbm, ibias_hbm,
                    inter_out, bias_out,
                    uidx_v, iidx_v, urows, irows, ubv, ibv, sem):
    wid = lax.axis_index("s") * NC + lax.axis_index("c")
    base = wid * BPW
    pltpu.sync_copy(uid_hbm.at[pl.ds(base, BPW)], uidx_v)
    pltpu.sync_copy(iid_hbm.at[pl.ds(base, BPW)], iidx_v)
    cu = pltpu.async_copy(uemb_hbm.at[uidx_v], urows, sem)
    ci = pltpu.async_copy(iemb_hbm.at[iidx_v], irows, sem)
    cub = pltpu.async_copy(ubias_hbm.at[uidx_v], ubv, sem)
    cib = pltpu.async_copy(ibias_hbm.at[iidx_v], ibv, sem)
    cu.wait()
    ci.wait()
    cub.wait()
    cib.wait()

    def row_body(r, carry):
        for c in range(D // L):
            sl = pl.ds(c * L, L)
            urows[r, sl] = urows[r, sl] * irows[r, sl]
        return carry

    lax.fori_loop(0, BPW, row_body, 0)
    for k in range(BPW // L):
        sl = pl.ds(k * L, L)
        ubv[sl] = ubv[sl] + ibv[sl]
    pltpu.sync_copy(urows, inter_out.at[pl.ds(base, BPW)])
    pltpu.sync_copy(ubv, bias_out.at[pl.ds(base, BPW)])


_sc_gather = pl.kernel(
    _sc_gather_body,
    mesh=_mesh,
    out_type=_SC_OUT_TYPE,
    scratch_types=_SC_SCRATCH,
)


def _tc_body(inter_ref, bias_ref, gb_ref, w1_ref, b1_ref, w2_ref,
             b2_ref, w3_ref, b3_ref, out_ref):
    x = inter_ref[...]
    cf = jnp.sum(x, axis=1, keepdims=True)
    cf = cf + bias_ref[...].reshape(-1, 1) + gb_ref[0, 0]
    h = jnp.maximum(jnp.dot(x, w1_ref[...], preferred_element_type=jnp.float32)
                    + b1_ref[...], 0.0)
    h = jnp.maximum(jnp.dot(h, w2_ref[...], preferred_element_type=jnp.float32)
                    + b2_ref[...], 0.0)
    mlp = jnp.dot(h, w3_ref[...], preferred_element_type=jnp.float32) + b3_ref[...]
    out_ref[...] = (cf + mlp).reshape(1, 1, -1)


RB = 1024
_GRID = B // RB


def _tc_call(inter, biasg, global_bias, W1, b1, W2, b2, W3, b3):
    rep = lambda i: (0, 0)
    out = pl.pallas_call(
        _tc_body,
        grid=(_GRID,),
        in_specs=[
            pl.BlockSpec((RB, D), lambda i: (i, 0)),
            pl.BlockSpec((1, 1, RB), lambda i: (i, 0, 0)),
            pl.BlockSpec((1, 1), rep),
            pl.BlockSpec((D, 128), rep),
            pl.BlockSpec((1, 128), rep),
            pl.BlockSpec((128, 64), rep),
            pl.BlockSpec((1, 64), rep),
            pl.BlockSpec((64, 1), rep),
            pl.BlockSpec((1, 1), rep),
        ],
        out_specs=pl.BlockSpec((1, 1, RB), lambda i: (i, 0, 0)),
        out_shape=jax.ShapeDtypeStruct((_GRID, 1, RB), jnp.float32),
    )(inter, biasg.reshape(_GRID, 1, RB), global_bias.reshape(1, 1), W1,
      b1.reshape(1, 128), W2, b2.reshape(1, 64), W3, b3.reshape(1, 1))
    return out.reshape(B)


def kernel(user_ids, pos_item_ids, user_emb, item_emb, user_bias, item_bias,
           global_bias, W1, b1, W2, b2, W3, b3):
    inter, biasg = _sc_gather(
        user_ids.astype(jnp.int32), pos_item_ids.astype(jnp.int32),
        user_emb, item_emb,
        user_bias.reshape(-1), item_bias.reshape(-1))
    return _tc_call(inter, biasg, global_bias, W1, b1, W2, b2, W3, b3)
